# Initial kernel scaffold; baseline (speedup 1.0000x reference)
import jax
import jax.numpy as jnp
from jax.experimental import pallas as pl


def kernel(atomic_embedding, pair_indices, f_ij, f_ij_cutoff, W_in, Wf1, bf1, Wf2, bf2, W2, b2, W3, b3):
    raise NotImplementedError("write your pallas kernel here")


# speedup vs baseline: 1.7331x; 1.7331x over previous
import functools

import jax
import jax.numpy as jnp
from jax import lax
from jax.experimental import pallas as pl
from jax.experimental.pallas import tpu as pltpu
from jax.experimental.pallas import tpu_sc as plsc

N_ATOMS = 10000
N_PAIRS = 320000
F = 128
N_RBF = 20

_LOG2 = 0.6931471805599453

_NC = 2
_NS = 16
_NW = _NC * _NS
_C = 80
_EPW = N_PAIRS // _NW
_NCH = _EPW // _C
_RPS = 624
_TAIL = N_ATOMS - _NS * _RPS
_ZR = 156

_HI = jax.lax.Precision.HIGHEST


def _ssp(x):
    return jnp.maximum(x, 0.0) + jnp.log1p(jnp.exp(-jnp.abs(x))) - _LOG2


def _mm(a, b):
    return jax.lax.dot_general(a, b, (((1,), (0,)), ((), ())),
                               precision=_HI, preferred_element_type=jnp.float32)


def _emb_body(a_ref, w_ref, o_ref):
    o_ref[...] = _mm(a_ref[...], w_ref[...])


def _filter_body(f_ref, cut_ref, wf1_ref, bf1_ref, wf2_ref, bf2_ref, o_ref):
    h = _ssp(_mm(f_ref[...], wf1_ref[...]) + bf1_ref[...])
    o_ref[...] = (_mm(h, wf2_ref[...]) + bf2_ref[...]) * cut_ref[...]


def _out_body(p0_ref, p1_ref, w2_ref, b2_ref, w3_ref, b3_ref, o_ref):
    s = p0_ref[...] + p1_ref[...]
    h = _ssp(_mm(s, w2_ref[...]) + b2_ref[...])
    o_ref[...] = _mm(h, w3_ref[...]) + b3_ref[...]


def _sc_edge_kernel(emb, wij, idx_i, idx_j):
    mesh = plsc.VectorSubcoreMesh(core_axis_name="c", subcore_axis_name="s")

    @functools.partial(
        pl.kernel,
        out_type=jax.ShapeDtypeStruct((_NC, N_ATOMS, F), jnp.float32),
        mesh=mesh,
        scratch_types=[
            pltpu.VMEM((_C,), jnp.int32),
            pltpu.VMEM((_C,), jnp.int32),
            pltpu.VMEM((_C, F), jnp.float32),
            pltpu.VMEM((_C, F), jnp.float32),
            pltpu.VMEM((_ZR, F), jnp.float32),
            pltpu.VMEM_SHARED((N_ATOMS, F), jnp.float32),
            pltpu.SemaphoreType.DMA,
            pltpu.SemaphoreType.DMA,
        ],
    )
    def k(emb_hbm, wij_hbm, idxi_hbm, idxj_hbm, out_hbm,
          idxj_v, idxi_v, xj_v, w_v, z_v, acc, sem_g, sem_w):
        cid = lax.axis_index("c")
        sid = lax.axis_index("s")

        @pl.loop(0, _ZR)
        def _(r):
            for c in range(F // 16):
                z_v[r, pl.ds(c * 16, 16)] = jnp.zeros((16,), jnp.float32)

        @pl.loop(0, _RPS, step=_ZR)
        def _(r0):
            pltpu.sync_copy(z_v, acc.at[pl.ds(sid * _RPS + r0, _ZR)])

        @pl.when(sid == 0)
        def _():
            pltpu.sync_copy(z_v.at[pl.ds(0, _TAIL)],
                            acc.at[pl.ds(_NS * _RPS, _TAIL)])

        plsc.subcore_barrier()

        wid = cid * _NS + sid
        base0 = wid * _EPW

        @pl.loop(0, _NCH)
        def _(ch):
            base = base0 + ch * _C
            pltpu.sync_copy(idxj_hbm.at[pl.ds(base, _C)], idxj_v)
            pltpu.sync_copy(idxi_hbm.at[pl.ds(base, _C)], idxi_v)
            g = pltpu.async_copy(emb_hbm.at[idxj_v], xj_v, sem_g)
            w = pltpu.async_copy(wij_hbm.at[pl.ds(base, _C)], w_v, sem_w)
            g.wait()
            w.wait()

            @pl.loop(0, _C)
            def _(e):
                for c in range(F // 16):
                    s = pl.ds(c * 16, 16)
                    xj_v[e, s] = xj_v[e, s] * w_v[e, s]

            pltpu.sync_copy(xj_v, acc.at[idxi_v], add=True)

        plsc.subcore_barrier()

        pltpu.sync_copy(acc.at[pl.ds(sid * _RPS, _RPS)],
                        out_hbm.at[cid, pl.ds(sid * _RPS, _RPS)])

        @pl.when(sid == 0)
        def _():
            pltpu.sync_copy(acc.at[pl.ds(_NS * _RPS, _TAIL)],
                            out_hbm.at[cid, pl.ds(_NS * _RPS, _TAIL)])

    return k(emb, wij, idx_i, idx_j)


def kernel(atomic_embedding, pair_indices, f_ij, f_ij_cutoff,
           W_in, Wf1, bf1, Wf2, bf2, W2, b2, W3, b3):
    idx_i = pair_indices[0]
    idx_j = pair_indices[1]

    emb = pl.pallas_call(
        _emb_body,
        out_shape=jax.ShapeDtypeStruct((N_ATOMS, F), jnp.float32),
    )(atomic_embedding, W_in)

    EB = 8000
    grid = (N_PAIRS // EB,)
    wij = pl.pallas_call(
        _filter_body,
        grid=grid,
        in_specs=[
            pl.BlockSpec((EB, N_RBF), lambda i: (i, 0)),
            pl.BlockSpec((EB, 1), lambda i: (i, 0)),
            pl.BlockSpec((N_RBF, F), lambda i: (0, 0)),
            pl.BlockSpec((1, F), lambda i: (0, 0)),
            pl.BlockSpec((F, F), lambda i: (0, 0)),
            pl.BlockSpec((1, F), lambda i: (0, 0)),
        ],
        out_specs=pl.BlockSpec((EB, F), lambda i: (i, 0)),
        out_shape=jax.ShapeDtypeStruct((N_PAIRS, F), jnp.float32),
    )(f_ij, f_ij_cutoff, Wf1, bf1.reshape(1, F), Wf2, bf2.reshape(1, F))

    partials = _sc_edge_kernel(emb, wij, idx_i, idx_j)

    NB = 2000
    out = pl.pallas_call(
        _out_body,
        grid=(N_ATOMS // NB,),
        in_specs=[
            pl.BlockSpec((NB, F), lambda i: (i, 0)),
            pl.BlockSpec((NB, F), lambda i: (i, 0)),
            pl.BlockSpec((F, F), lambda i: (0, 0)),
            pl.BlockSpec((1, F), lambda i: (0, 0)),
            pl.BlockSpec((F, F), lambda i: (0, 0)),
            pl.BlockSpec((1, F), lambda i: (0, 0)),
        ],
        out_specs=pl.BlockSpec((NB, F), lambda i: (i, 0)),
        out_shape=jax.ShapeDtypeStruct((N_ATOMS, F), jnp.float32),
    )(partials[0], partials[1], W2, b2.reshape(1, F), W3, b3.reshape(1, F))

    return out


# speedup vs baseline: 2.2119x; 1.2763x over previous
import functools

import jax
import jax.numpy as jnp
from jax import lax
from jax.experimental import pallas as pl
from jax.experimental.pallas import tpu as pltpu
from jax.experimental.pallas import tpu_sc as plsc

N_ATOMS = 10000
N_PAIRS = 320000
F = 128
N_RBF = 20

_LOG2 = 0.6931471805599453

_NC = 2
_NS = 16
_NW = _NC * _NS
_C = 80
_EPW = N_PAIRS // _NW
_NCH = _EPW // _C
_RPS = 624
_TAIL = N_ATOMS - _NS * _RPS
_ZR = 156

def _ssp(x):
    return jnp.maximum(x, 0.0) + jnp.log1p(jnp.exp(-jnp.abs(x))) - _LOG2


def _dot(a, b):
    return jax.lax.dot_general(a, b, (((1,), (0,)), ((), ())),
                               preferred_element_type=jnp.float32)


def _mm(a, b):
    ah = a.astype(jnp.bfloat16)
    al = (a - ah.astype(jnp.float32)).astype(jnp.bfloat16)
    bh = b.astype(jnp.bfloat16)
    bl = (b - bh.astype(jnp.float32)).astype(jnp.bfloat16)
    return _dot(ah, bh) + _dot(ah, bl) + _dot(al, bh)


def _emb_body(a_ref, w_ref, o_ref):
    o_ref[...] = _mm(a_ref[...], w_ref[...])


def _filter_body(f_ref, cut_ref, wf1_ref, bf1_ref, wf2_ref, bf2_ref, o_ref):
    h = _ssp(_mm(f_ref[...], wf1_ref[...]) + bf1_ref[...])
    o_ref[...] = (_mm(h, wf2_ref[...]) + bf2_ref[...]) * cut_ref[...]


def _out_body(p0_ref, p1_ref, w2_ref, b2_ref, w3_ref, b3_ref, o_ref):
    s = p0_ref[...] + p1_ref[...]
    h = _ssp(_mm(s, w2_ref[...]) + b2_ref[...])
    o_ref[...] = _mm(h, w3_ref[...]) + b3_ref[...]


def _sc_edge_kernel(emb, wij, idx_i, idx_j):
    mesh = plsc.VectorSubcoreMesh(core_axis_name="c", subcore_axis_name="s")

    @functools.partial(
        pl.kernel,
        out_type=jax.ShapeDtypeStruct((_NC, N_ATOMS, F), jnp.float32),
        mesh=mesh,
        scratch_types=[
            pltpu.VMEM((_C,), jnp.int32),
            pltpu.VMEM((_C,), jnp.int32),
            pltpu.VMEM((_C, F), jnp.float32),
            pltpu.VMEM((_C, F), jnp.float32),
            pltpu.VMEM((_ZR, F), jnp.float32),
            pltpu.VMEM_SHARED((N_ATOMS, F), jnp.float32),
            pltpu.SemaphoreType.DMA,
            pltpu.SemaphoreType.DMA,
        ],
    )
    def k(emb_hbm, wij_hbm, idxi_hbm, idxj_hbm, out_hbm,
          idxj_v, idxi_v, xj_v, w_v, z_v, acc, sem_g, sem_w):
        cid = lax.axis_index("c")
        sid = lax.axis_index("s")

        @pl.loop(0, _ZR)
        def _(r):
            for c in range(F // 16):
                z_v[r, pl.ds(c * 16, 16)] = jnp.zeros((16,), jnp.float32)

        @pl.loop(0, _RPS, step=_ZR)
        def _(r0):
            pltpu.sync_copy(z_v, acc.at[pl.ds(sid * _RPS + r0, _ZR)])

        @pl.when(sid == 0)
        def _():
            pltpu.sync_copy(z_v.at[pl.ds(0, _TAIL)],
                            acc.at[pl.ds(_NS * _RPS, _TAIL)])

        plsc.subcore_barrier()

        wid = cid * _NS + sid
        base0 = wid * _EPW

        @pl.loop(0, _NCH)
        def _(ch):
            base = base0 + ch * _C
            pltpu.sync_copy(idxj_hbm.at[pl.ds(base, _C)], idxj_v)
            pltpu.sync_copy(idxi_hbm.at[pl.ds(base, _C)], idxi_v)
            g = pltpu.async_copy(emb_hbm.at[idxj_v], xj_v, sem_g)
            w = pltpu.async_copy(wij_hbm.at[pl.ds(base, _C)], w_v, sem_w)
            g.wait()
            w.wait()

            @pl.loop(0, _C)
            def _(e):
                for c in range(F // 16):
                    s = pl.ds(c * 16, 16)
                    xj_v[e, s] = xj_v[e, s] * w_v[e, s]

            pltpu.sync_copy(xj_v, acc.at[idxi_v], add=True)

        plsc.subcore_barrier()

        pltpu.sync_copy(acc.at[pl.ds(sid * _RPS, _RPS)],
                        out_hbm.at[cid, pl.ds(sid * _RPS, _RPS)])

        @pl.when(sid == 0)
        def _():
            pltpu.sync_copy(acc.at[pl.ds(_NS * _RPS, _TAIL)],
                            out_hbm.at[cid, pl.ds(_NS * _RPS, _TAIL)])

    return k(emb, wij, idx_i, idx_j)


def kernel(atomic_embedding, pair_indices, f_ij, f_ij_cutoff,
           W_in, Wf1, bf1, Wf2, bf2, W2, b2, W3, b3):
    idx_i = pair_indices[0]
    idx_j = pair_indices[1]

    emb = pl.pallas_call(
        _emb_body,
        out_shape=jax.ShapeDtypeStruct((N_ATOMS, F), jnp.float32),
    )(atomic_embedding, W_in)

    EB = 8000
    grid = (N_PAIRS // EB,)
    wij = pl.pallas_call(
        _filter_body,
        grid=grid,
        in_specs=[
            pl.BlockSpec((EB, N_RBF), lambda i: (i, 0)),
            pl.BlockSpec((EB, 1), lambda i: (i, 0)),
            pl.BlockSpec((N_RBF, F), lambda i: (0, 0)),
            pl.BlockSpec((1, F), lambda i: (0, 0)),
            pl.BlockSpec((F, F), lambda i: (0, 0)),
            pl.BlockSpec((1, F), lambda i: (0, 0)),
        ],
        out_specs=pl.BlockSpec((EB, F), lambda i: (i, 0)),
        out_shape=jax.ShapeDtypeStruct((N_PAIRS, F), jnp.float32),
    )(f_ij, f_ij_cutoff, Wf1, bf1.reshape(1, F), Wf2, bf2.reshape(1, F))

    partials = _sc_edge_kernel(emb, wij, idx_i, idx_j)

    NB = 2000
    out = pl.pallas_call(
        _out_body,
        grid=(N_ATOMS // NB,),
        in_specs=[
            pl.BlockSpec((NB, F), lambda i: (i, 0)),
            pl.BlockSpec((NB, F), lambda i: (i, 0)),
            pl.BlockSpec((F, F), lambda i: (0, 0)),
            pl.BlockSpec((1, F), lambda i: (0, 0)),
            pl.BlockSpec((F, F), lambda i: (0, 0)),
            pl.BlockSpec((1, F), lambda i: (0, 0)),
        ],
        out_specs=pl.BlockSpec((NB, F), lambda i: (i, 0)),
        out_shape=jax.ShapeDtypeStruct((N_ATOMS, F), jnp.float32),
    )(partials[0], partials[1], W2, b2.reshape(1, F), W3, b3.reshape(1, F))

    return out


# speedup vs baseline: 2.9682x; 1.3419x over previous
import functools

import jax
import jax.numpy as jnp
from jax import lax
from jax.experimental import pallas as pl
from jax.experimental.pallas import tpu as pltpu
from jax.experimental.pallas import tpu_sc as plsc

N_ATOMS = 10000
N_PAIRS = 320000
F = 128
N_RBF = 20

_LOG2 = 0.6931471805599453

_NC = 2
_NS = 16
_NW = _NC * _NS
_C = 40
_EPW = N_PAIRS // _NW
_NCH = _EPW // _C
_RPS = 624
_TAIL = N_ATOMS - _NS * _RPS

def _ssp(x):
    return jnp.maximum(x, 0.0) + jnp.log1p(jnp.exp(-jnp.abs(x))) - _LOG2


def _dot(a, b):
    return jax.lax.dot_general(a, b, (((1,), (0,)), ((), ())),
                               preferred_element_type=jnp.float32)


def _mm(a, b):
    ah = a.astype(jnp.bfloat16)
    al = (a - ah.astype(jnp.float32)).astype(jnp.bfloat16)
    bh = b.astype(jnp.bfloat16)
    bl = (b - bh.astype(jnp.float32)).astype(jnp.bfloat16)
    return _dot(ah, bh) + _dot(ah, bl) + _dot(al, bh)


def _emb_body(a_ref, w_ref, o_ref):
    o_ref[...] = _mm(a_ref[...], w_ref[...])


def _filter_body(f_ref, cut_ref, wf1_ref, bf1_ref, wf2_ref, bf2_ref, o_ref):
    h = _ssp(_mm(f_ref[...], wf1_ref[...]) + bf1_ref[...])
    o_ref[...] = (_mm(h, wf2_ref[...]) + bf2_ref[...]) * cut_ref[...]


def _out_body(p0_ref, p1_ref, w2_ref, b2_ref, w3_ref, b3_ref, o_ref):
    s = p0_ref[...] + p1_ref[...]
    h = _ssp(_mm(s, w2_ref[...]) + b2_ref[...])
    o_ref[...] = _mm(h, w3_ref[...]) + b3_ref[...]


def _sc_edge_kernel(emb, wij, idx_i3, idx_j3):
    mesh = plsc.VectorSubcoreMesh(core_axis_name="c", subcore_axis_name="s")

    @functools.partial(
        pl.kernel,
        out_type=jax.ShapeDtypeStruct((_NC, N_ATOMS, F), jnp.float32),
        mesh=mesh,
        scratch_types=[
            pltpu.VMEM((_EPW,), jnp.int32),
            pltpu.VMEM((_EPW,), jnp.int32),
            pltpu.VMEM((_C, F), jnp.float32),
            pltpu.VMEM((_C, F), jnp.float32),
            pltpu.VMEM((_C, F), jnp.float32),
            pltpu.VMEM((_C, F), jnp.float32),
            pltpu.VMEM((_C, F), jnp.float32),
            pltpu.VMEM((_C, F), jnp.float32),
            pltpu.VMEM_SHARED((N_ATOMS, F), jnp.float32),
            pltpu.SemaphoreType.DMA,
            pltpu.SemaphoreType.DMA,
            pltpu.SemaphoreType.DMA,
            pltpu.SemaphoreType.DMA,
            pltpu.SemaphoreType.DMA,
            pltpu.SemaphoreType.DMA,
        ],
    )
    def k(emb_hbm, wij_hbm, idxi_hbm, idxj_hbm, out_hbm,
          idxj_t, idxi_t, xj0, xj1, w0, w1, pr0, pr1, acc,
          sg0, sg1, sw0, sw1, ss0, ss1):
        xj = (xj0, xj1)
        wv = (w0, w1)
        pr = (pr0, pr1)
        sg = (sg0, sg1)
        sw = (sw0, sw1)
        ss = (ss0, ss1)

        cid = lax.axis_index("c")
        sid = lax.axis_index("s")
        wid = cid * _NS + sid
        base0 = wid * _EPW

        pltpu.sync_copy(idxj_hbm.at[pl.ds(base0, _EPW)], idxj_t)
        pltpu.sync_copy(idxi_hbm.at[pl.ds(base0, _EPW)], idxi_t)

        def issue_loads(c, p):
            pltpu.async_copy(emb_hbm.at[idxj_t.at[pl.ds(c * _C, _C)]],
                             xj[p], sg[p])
            pltpu.async_copy(wij_hbm.at[pl.ds(base0 + c * _C, _C)], wv[p], sw[p])

        issue_loads(0, 0)
        issue_loads(1, 1)

        @pl.loop(0, _C)
        def _(r):
            for c in range(F // 16):
                pr0[r, pl.ds(c * 16, 16)] = jnp.zeros((16,), jnp.float32)

        @pl.loop(0, _RPS - _C, step=_C)
        def _(r0):
            pltpu.sync_copy(pr0, acc.at[pl.ds(sid * _RPS + r0, _C)])

        rem = _RPS % _C if _RPS % _C else _C
        pltpu.sync_copy(pr0.at[pl.ds(0, rem)],
                        acc.at[pl.ds(sid * _RPS + _RPS - rem, rem)])

        @pl.when(sid == 0)
        def _():
            pltpu.sync_copy(pr0.at[pl.ds(0, _TAIL)],
                            acc.at[pl.ds(_NS * _RPS, _TAIL)])

        plsc.subcore_barrier()

        def process(c, p, prefetch):
            pltpu.make_async_copy(
                emb_hbm.at[idxj_t.at[pl.ds(c * _C, _C)]], xj[p], sg[p]).wait()
            pltpu.make_async_copy(
                wij_hbm.at[pl.ds(base0 + c * _C, _C)], wv[p], sw[p]).wait()

            @pl.when(c >= 2)
            def _():
                pltpu.make_async_copy(
                    pr[p], acc.at[idxi_t.at[pl.ds(c * _C, _C)]], ss[p]).wait()

            @pl.loop(0, _C)
            def _(e):
                for col in range(F // 16):
                    s = pl.ds(col * 16, 16)
                    pr[p][e, s] = xj[p][e, s] * wv[p][e, s]

            pltpu.async_copy(pr[p], acc.at[idxi_t.at[pl.ds(c * _C, _C)]],
                             ss[p], add=True)
            if prefetch:
                @pl.when(c + 2 < _NCH)
                def _():
                    issue_loads(c + 2, p)

        @pl.loop(0, _NCH, step=2)
        def _(ch):
            process(ch, 0, True)
            process(ch + 1, 1, True)

        pltpu.make_async_copy(pr[0], acc.at[idxi_t.at[pl.ds(0, _C)]], ss[0]).wait()
        pltpu.make_async_copy(pr[1], acc.at[idxi_t.at[pl.ds(0, _C)]], ss[1]).wait()

        plsc.subcore_barrier()

        pltpu.sync_copy(acc.at[pl.ds(sid * _RPS, _RPS)],
                        out_hbm.at[cid, pl.ds(sid * _RPS, _RPS)])

        @pl.when(sid == 0)
        def _():
            pltpu.sync_copy(acc.at[pl.ds(_NS * _RPS, _TAIL)],
                            out_hbm.at[cid, pl.ds(_NS * _RPS, _TAIL)])

    return k(emb, wij, idx_i3, idx_j3)


def kernel(atomic_embedding, pair_indices, f_ij, f_ij_cutoff,
           W_in, Wf1, bf1, Wf2, bf2, W2, b2, W3, b3):
    idx_i = pair_indices[0]
    idx_j = pair_indices[1]

    emb = pl.pallas_call(
        _emb_body,
        out_shape=jax.ShapeDtypeStruct((N_ATOMS, F), jnp.float32),
    )(atomic_embedding, W_in)

    EB = 8000
    grid = (N_PAIRS // EB,)
    wij = pl.pallas_call(
        _filter_body,
        grid=grid,
        in_specs=[
            pl.BlockSpec((EB, N_RBF), lambda i: (i, 0)),
            pl.BlockSpec((EB, 1), lambda i: (i, 0)),
            pl.BlockSpec((N_RBF, F), lambda i: (0, 0)),
            pl.BlockSpec((1, F), lambda i: (0, 0)),
            pl.BlockSpec((F, F), lambda i: (0, 0)),
            pl.BlockSpec((1, F), lambda i: (0, 0)),
        ],
        out_specs=pl.BlockSpec((EB, F), lambda i: (i, 0)),
        out_shape=jax.ShapeDtypeStruct((N_PAIRS, F), jnp.float32),
    )(f_ij, f_ij_cutoff, Wf1, bf1.reshape(1, F), Wf2, bf2.reshape(1, F))

    partials = _sc_edge_kernel(emb, wij, idx_i, idx_j)

    NB = 2000
    out = pl.pallas_call(
        _out_body,
        grid=(N_ATOMS // NB,),
        in_specs=[
            pl.BlockSpec((NB, F), lambda i: (i, 0)),
            pl.BlockSpec((NB, F), lambda i: (i, 0)),
            pl.BlockSpec((F, F), lambda i: (0, 0)),
            pl.BlockSpec((1, F), lambda i: (0, 0)),
            pl.BlockSpec((F, F), lambda i: (0, 0)),
            pl.BlockSpec((1, F), lambda i: (0, 0)),
        ],
        out_specs=pl.BlockSpec((NB, F), lambda i: (i, 0)),
        out_shape=jax.ShapeDtypeStruct((N_ATOMS, F), jnp.float32),
    )(partials[0], partials[1], W2, b2.reshape(1, F), W3, b3.reshape(1, F))

    return out


# speedup vs baseline: 3.5783x; 1.2055x over previous
import functools

import jax
import jax.numpy as jnp
from jax import lax
from jax.experimental import pallas as pl
from jax.experimental.pallas import tpu as pltpu
from jax.experimental.pallas import tpu_sc as plsc

N_ATOMS = 10000
N_PAIRS = 320000
F = 128
N_RBF = 20

_LOG2 = 0.6931471805599453

_NC = 2
_NS = 16
_NW = _NC * _NS
_C = 40
_EPW = N_PAIRS // _NW
_NCH = _EPW // _C
_RPS = 624
_TAIL = N_ATOMS - _NS * _RPS

def _ssp(x):
    return jnp.maximum(x, 0.0) + jnp.log1p(jnp.exp(-jnp.abs(x))) - _LOG2


def _dot(a, b):
    return jax.lax.dot_general(a, b, (((1,), (0,)), ((), ())),
                               preferred_element_type=jnp.float32)


def _mm(a, b):
    ah = a.astype(jnp.bfloat16)
    al = (a - ah.astype(jnp.float32)).astype(jnp.bfloat16)
    bh = b.astype(jnp.bfloat16)
    bl = (b - bh.astype(jnp.float32)).astype(jnp.bfloat16)
    return _dot(ah, bh) + _dot(ah, bl) + _dot(al, bh)


def _emb_body(a_ref, w_ref, o_ref):
    o_ref[...] = _mm(a_ref[...], w_ref[...])


def _filter_body(f_ref, cut_ref, wf1_ref, bf1_ref, wf2_ref, bf2_ref, o_ref):
    h = _ssp(_mm(f_ref[...], wf1_ref[...]) + bf1_ref[...])
    o_ref[...] = (_dot(h.astype(jnp.bfloat16), wf2_ref[...].astype(jnp.bfloat16))
                  + bf2_ref[...]) * cut_ref[...]


def _out_body(*refs):
    p_refs = refs[:-5]
    w2_ref, b2_ref, w3_ref, b3_ref, o_ref = refs[-5:]
    s = p_refs[0][...]
    for p_ref in p_refs[1:]:
        s = s + p_ref[...]
    h = _ssp(_mm(s, w2_ref[...]) + b2_ref[...])
    o_ref[...] = _mm(h, w3_ref[...]) + b3_ref[...]


def _sc_edge_kernel(emb, wij, idx_i, idx_j, epw):
    nch = epw // _C
    mesh = plsc.VectorSubcoreMesh(core_axis_name="c", subcore_axis_name="s")

    @functools.partial(
        pl.kernel,
        out_type=jax.ShapeDtypeStruct((_NC, N_ATOMS, F), jnp.float32),
        mesh=mesh,
        scratch_types=[
            pltpu.VMEM((epw,), jnp.int32),
            pltpu.VMEM((epw,), jnp.int32),
            pltpu.VMEM((_C, F), jnp.float32),
            pltpu.VMEM((_C, F), jnp.float32),
            pltpu.VMEM((_C, F), jnp.float32),
            pltpu.VMEM((_C, F), jnp.float32),
            pltpu.VMEM((_C, F), jnp.float32),
            pltpu.VMEM((_C, F), jnp.float32),
            pltpu.VMEM_SHARED((N_ATOMS, F), jnp.float32),
            pltpu.SemaphoreType.DMA,
            pltpu.SemaphoreType.DMA,
            pltpu.SemaphoreType.DMA,
            pltpu.SemaphoreType.DMA,
            pltpu.SemaphoreType.DMA,
            pltpu.SemaphoreType.DMA,
        ],
    )
    def k(emb_hbm, wij_hbm, idxi_hbm, idxj_hbm, out_hbm,
          idxj_t, idxi_t, xj0, xj1, w0, w1, pr0, pr1, acc,
          sg0, sg1, sw0, sw1, ss0, ss1):
        xj = (xj0, xj1)
        wv = (w0, w1)
        pr = (pr0, pr1)
        sg = (sg0, sg1)
        sw = (sw0, sw1)
        ss = (ss0, ss1)

        cid = lax.axis_index("c")
        sid = lax.axis_index("s")
        wid = cid * _NS + sid
        base0 = wid * epw

        pltpu.sync_copy(idxj_hbm.at[pl.ds(base0, epw)], idxj_t)
        pltpu.sync_copy(idxi_hbm.at[pl.ds(base0, epw)], idxi_t)

        def issue_loads(c, p):
            pltpu.async_copy(emb_hbm.at[idxj_t.at[pl.ds(c * _C, _C)]],
                             xj[p], sg[p])
            pltpu.async_copy(wij_hbm.at[pl.ds(base0 + c * _C, _C)], wv[p], sw[p])

        issue_loads(0, 0)
        issue_loads(1, 1)

        @pl.loop(0, _C)
        def _(r):
            for c in range(F // 16):
                pr0[r, pl.ds(c * 16, 16)] = jnp.zeros((16,), jnp.float32)

        @pl.loop(0, _RPS - _C, step=_C)
        def _(r0):
            pltpu.sync_copy(pr0, acc.at[pl.ds(sid * _RPS + r0, _C)])

        rem = _RPS % _C if _RPS % _C else _C
        pltpu.sync_copy(pr0.at[pl.ds(0, rem)],
                        acc.at[pl.ds(sid * _RPS + _RPS - rem, rem)])

        @pl.when(sid == 0)
        def _():
            pltpu.sync_copy(pr0.at[pl.ds(0, _TAIL)],
                            acc.at[pl.ds(_NS * _RPS, _TAIL)])

        plsc.subcore_barrier()

        def process(c, p, prefetch):
            pltpu.make_async_copy(
                emb_hbm.at[idxj_t.at[pl.ds(c * _C, _C)]], xj[p], sg[p]).wait()
            pltpu.make_async_copy(
                wij_hbm.at[pl.ds(base0 + c * _C, _C)], wv[p], sw[p]).wait()

            @pl.when(c >= 2)
            def _():
                pltpu.make_async_copy(
                    pr[p], acc.at[idxi_t.at[pl.ds(c * _C, _C)]], ss[p]).wait()

            @pl.loop(0, _C)
            def _(e):
                for col in range(F // 16):
                    s = pl.ds(col * 16, 16)
                    pr[p][e, s] = xj[p][e, s] * wv[p][e, s]

            pltpu.async_copy(pr[p], acc.at[idxi_t.at[pl.ds(c * _C, _C)]],
                             ss[p], add=True)
            if prefetch:
                @pl.when(c + 2 < nch)
                def _():
                    issue_loads(c + 2, p)

        @pl.loop(0, nch - (nch % 2), step=2)
        def _(ch):
            process(ch, 0, True)
            process(ch + 1, 1, True)

        if nch % 2:
            process(nch - 1, 0, False)

        pltpu.make_async_copy(pr[0], acc.at[idxi_t.at[pl.ds(0, _C)]], ss[0]).wait()
        pltpu.make_async_copy(pr[1], acc.at[idxi_t.at[pl.ds(0, _C)]], ss[1]).wait()

        plsc.subcore_barrier()

        pltpu.sync_copy(acc.at[pl.ds(sid * _RPS, _RPS)],
                        out_hbm.at[cid, pl.ds(sid * _RPS, _RPS)])

        @pl.when(sid == 0)
        def _():
            pltpu.sync_copy(acc.at[pl.ds(_NS * _RPS, _TAIL)],
                            out_hbm.at[cid, pl.ds(_NS * _RPS, _TAIL)])

    return k(emb, wij, idx_i, idx_j)


def _filter_block(f_ij, f_ij_cutoff, Wf1, bf1, Wf2, bf2, eb):
    e = f_ij.shape[0]
    return pl.pallas_call(
        _filter_body,
        grid=(e // eb,),
        in_specs=[
            pl.BlockSpec((eb, N_RBF), lambda i: (i, 0)),
            pl.BlockSpec((eb, 1), lambda i: (i, 0)),
            pl.BlockSpec((N_RBF, F), lambda i: (0, 0)),
            pl.BlockSpec((1, F), lambda i: (0, 0)),
            pl.BlockSpec((F, F), lambda i: (0, 0)),
            pl.BlockSpec((1, F), lambda i: (0, 0)),
        ],
        out_specs=pl.BlockSpec((eb, F), lambda i: (i, 0)),
        out_shape=jax.ShapeDtypeStruct((e, F), jnp.float32),
    )(f_ij, f_ij_cutoff, Wf1, bf1.reshape(1, F), Wf2, bf2.reshape(1, F))


_NSPLIT = 2


def kernel(atomic_embedding, pair_indices, f_ij, f_ij_cutoff,
           W_in, Wf1, bf1, Wf2, bf2, W2, b2, W3, b3):
    idx_i = pair_indices[0]
    idx_j = pair_indices[1]

    emb = pl.pallas_call(
        _emb_body,
        out_shape=jax.ShapeDtypeStruct((N_ATOMS, F), jnp.float32),
    )(atomic_embedding, W_in)

    es = N_PAIRS // _NSPLIT
    epw = es // _NW
    partials = []
    for s in range(_NSPLIT):
        sl = slice(s * es, (s + 1) * es)
        wij = _filter_block(f_ij[sl], f_ij_cutoff[sl], Wf1, bf1, Wf2, bf2, 8000)
        partials.append(_sc_edge_kernel(emb, wij, idx_i[sl], idx_j[sl], epw))

    NB = 2000
    nsum = 2 * _NSPLIT
    out = pl.pallas_call(
        _out_body,
        grid=(N_ATOMS // NB,),
        in_specs=[pl.BlockSpec((NB, F), lambda i: (i, 0))] * nsum + [
            pl.BlockSpec((F, F), lambda i: (0, 0)),
            pl.BlockSpec((1, F), lambda i: (0, 0)),
            pl.BlockSpec((F, F), lambda i: (0, 0)),
            pl.BlockSpec((1, F), lambda i: (0, 0)),
        ],
        out_specs=pl.BlockSpec((NB, F), lambda i: (i, 0)),
        out_shape=jax.ShapeDtypeStruct((N_ATOMS, F), jnp.float32),
    )(*(p[c] for p in partials for c in range(_NC)),
      W2, b2.reshape(1, F), W3, b3.reshape(1, F))

    return out


# speedup vs baseline: 3.7403x; 1.0453x over previous
import functools

import jax
import jax.numpy as jnp
from jax import lax
from jax.experimental import pallas as pl
from jax.experimental.pallas import tpu as pltpu
from jax.experimental.pallas import tpu_sc as plsc

N_ATOMS = 10000
N_PAIRS = 320000
F = 128
N_RBF = 20

_LOG2 = 0.6931471805599453

_NC = 2
_NS = 16
_NW = _NC * _NS
_C = 40
_RPS = 624
_TAIL = N_ATOMS - _NS * _RPS

_NSPLIT = 2
_ES = N_PAIRS // _NSPLIT
_EPW = _ES // _NW
_EB = 8000


def _ssp(x):
    return jnp.maximum(x, 0.0) + jnp.log1p(jnp.exp(-jnp.abs(x))) - _LOG2


def _dot(a, b):
    return jax.lax.dot_general(a, b, (((1,), (0,)), ((), ())),
                               preferred_element_type=jnp.float32)


def _mm(a, b):
    ah = a.astype(jnp.bfloat16)
    al = (a - ah.astype(jnp.float32)).astype(jnp.bfloat16)
    bh = b.astype(jnp.bfloat16)
    bl = (b - bh.astype(jnp.float32)).astype(jnp.bfloat16)
    return _dot(ah, bh) + _dot(ah, bl) + _dot(al, bh)


def _emb_body(a_ref, w_ref, o_ref):
    o_ref[...] = _mm(a_ref[...], w_ref[...])


def _filter_body(f_ref, cut_ref, wf1_ref, bf1_ref, wf2_ref, bf2_ref, o_ref):
    h = _ssp(_mm(f_ref[...], wf1_ref[...]) + bf1_ref[...])
    o_ref[...] = (_dot(h.astype(jnp.bfloat16), wf2_ref[...].astype(jnp.bfloat16))
                  + bf2_ref[...]) * cut_ref[...]


def _filter_block(f_ij, f_ij_cutoff, Wf1, bf1, Wf2, bf2, s):
    off = s * (_ES // _EB)
    return pl.pallas_call(
        _filter_body,
        grid=(_ES // _EB,),
        in_specs=[
            pl.BlockSpec((_EB, N_RBF), lambda i: (i + off, 0)),
            pl.BlockSpec((_EB, 1), lambda i: (i + off, 0)),
            pl.BlockSpec((N_RBF, F), lambda i: (0, 0)),
            pl.BlockSpec((1, F), lambda i: (0, 0)),
            pl.BlockSpec((F, F), lambda i: (0, 0)),
            pl.BlockSpec((1, F), lambda i: (0, 0)),
        ],
        out_specs=pl.BlockSpec((_EB, F), lambda i: (i, 0)),
        out_shape=jax.ShapeDtypeStruct((_ES, F), jnp.float32),
    )(f_ij, f_ij_cutoff, Wf1, bf1.reshape(1, F), Wf2, bf2.reshape(1, F))


def _out_body(*refs):
    p_refs = refs[:-5]
    w2_ref, b2_ref, w3_ref, b3_ref, o_ref = refs[-5:]
    s = p_refs[0][...]
    for p_ref in p_refs[1:]:
        s = s + p_ref[...]
    h = _ssp(_mm(s, w2_ref[...]) + b2_ref[...])
    o_ref[...] = _mm(h, w3_ref[...]) + b3_ref[...]


def _sc_edge_kernel(emb, wij, idx_i, idx_j, base_edge):
    epw = _EPW
    nch = epw // _C
    mesh = plsc.VectorSubcoreMesh(core_axis_name="c", subcore_axis_name="s")

    @functools.partial(
        pl.kernel,
        out_type=(jax.ShapeDtypeStruct((N_ATOMS, F), jnp.float32),
                  jax.ShapeDtypeStruct((N_ATOMS, F), jnp.float32)),
        mesh=mesh,
        scratch_types=[
            pltpu.VMEM((epw,), jnp.int32),
            pltpu.VMEM((epw,), jnp.int32),
            pltpu.VMEM((_C, F), jnp.float32),
            pltpu.VMEM((_C, F), jnp.float32),
            pltpu.VMEM((_C, F), jnp.float32),
            pltpu.VMEM((_C, F), jnp.float32),
            pltpu.VMEM((_C, F), jnp.float32),
            pltpu.VMEM((_C, F), jnp.float32),
            pltpu.VMEM_SHARED((N_ATOMS, F), jnp.float32),
            pltpu.SemaphoreType.DMA,
            pltpu.SemaphoreType.DMA,
            pltpu.SemaphoreType.DMA,
            pltpu.SemaphoreType.DMA,
            pltpu.SemaphoreType.DMA,
            pltpu.SemaphoreType.DMA,
        ],
    )
    def k(emb_hbm, wij_hbm, idxi_hbm, idxj_hbm, out0_hbm, out1_hbm,
          idxj_t, idxi_t, xj0, xj1, w0, w1, pr0, pr1, acc,
          sg0, sg1, sw0, sw1, ss0, ss1):
        xj = (xj0, xj1)
        wv = (w0, w1)
        pr = (pr0, pr1)
        sg = (sg0, sg1)
        sw = (sw0, sw1)
        ss = (ss0, ss1)

        cid = lax.axis_index("c")
        sid = lax.axis_index("s")
        wid = cid * _NS + sid
        base0 = wid * epw

        pltpu.sync_copy(idxj_hbm.at[pl.ds(base_edge + base0, epw)], idxj_t)
        pltpu.sync_copy(idxi_hbm.at[pl.ds(base_edge + base0, epw)], idxi_t)

        def issue_loads(c, p):
            pltpu.async_copy(emb_hbm.at[idxj_t.at[pl.ds(c * _C, _C)]],
                             xj[p], sg[p])
            pltpu.async_copy(wij_hbm.at[pl.ds(base0 + c * _C, _C)], wv[p], sw[p])

        issue_loads(0, 0)
        issue_loads(1, 1)

        @pl.loop(0, _C)
        def _(r):
            for c in range(F // 16):
                pr0[r, pl.ds(c * 16, 16)] = jnp.zeros((16,), jnp.float32)

        @pl.loop(0, _RPS - _C, step=_C)
        def _(r0):
            pltpu.sync_copy(pr0, acc.at[pl.ds(sid * _RPS + r0, _C)])

        rem = _RPS % _C if _RPS % _C else _C
        pltpu.sync_copy(pr0.at[pl.ds(0, rem)],
                        acc.at[pl.ds(sid * _RPS + _RPS - rem, rem)])

        @pl.when(sid == 0)
        def _():
            pltpu.sync_copy(pr0.at[pl.ds(0, _TAIL)],
                            acc.at[pl.ds(_NS * _RPS, _TAIL)])

        plsc.subcore_barrier()

        def process(c, p, prefetch):
            pltpu.make_async_copy(
                emb_hbm.at[idxj_t.at[pl.ds(c * _C, _C)]], xj[p], sg[p]).wait()
            pltpu.make_async_copy(
                wij_hbm.at[pl.ds(base0 + c * _C, _C)], wv[p], sw[p]).wait()

            @pl.when(c >= 2)
            def _():
                pltpu.make_async_copy(
                    pr[p], acc.at[idxi_t.at[pl.ds(c * _C, _C)]], ss[p]).wait()

            @pl.loop(0, _C)
            def _(e):
                for col in range(F // 16):
                    s = pl.ds(col * 16, 16)
                    pr[p][e, s] = xj[p][e, s] * wv[p][e, s]

            pltpu.async_copy(pr[p], acc.at[idxi_t.at[pl.ds(c * _C, _C)]],
                             ss[p], add=True)
            if prefetch:
                @pl.when(c + 2 < nch)
                def _():
                    issue_loads(c + 2, p)

        @pl.loop(0, nch - (nch % 2), step=2)
        def _(ch):
            process(ch, 0, True)
            process(ch + 1, 1, True)

        if nch % 2:
            process(nch - 1, 0, False)

        pltpu.make_async_copy(pr[0], acc.at[idxi_t.at[pl.ds(0, _C)]], ss[0]).wait()
        pltpu.make_async_copy(pr[1], acc.at[idxi_t.at[pl.ds(0, _C)]], ss[1]).wait()

        plsc.subcore_barrier()

        def dump(out_hbm):
            pltpu.sync_copy(acc.at[pl.ds(sid * _RPS, _RPS)],
                            out_hbm.at[pl.ds(sid * _RPS, _RPS)])

            @pl.when(sid == 0)
            def _():
                pltpu.sync_copy(acc.at[pl.ds(_NS * _RPS, _TAIL)],
                                out_hbm.at[pl.ds(_NS * _RPS, _TAIL)])

        @pl.when(cid == 0)
        def _():
            dump(out0_hbm)

        @pl.when(cid == 1)
        def _():
            dump(out1_hbm)

    return k(emb, wij, idx_i, idx_j)


def kernel(atomic_embedding, pair_indices, f_ij, f_ij_cutoff,
           W_in, Wf1, bf1, Wf2, bf2, W2, b2, W3, b3):
    idx_i = pair_indices[0]
    idx_j = pair_indices[1]

    emb = pl.pallas_call(
        _emb_body,
        out_shape=jax.ShapeDtypeStruct((N_ATOMS, F), jnp.float32),
    )(atomic_embedding, W_in)

    partials = []
    for s in range(_NSPLIT):
        wij = _filter_block(f_ij, f_ij_cutoff, Wf1, bf1, Wf2, bf2, s)
        partials.extend(_sc_edge_kernel(emb, wij, idx_i, idx_j, s * _ES))

    NB = 2000
    out = pl.pallas_call(
        _out_body,
        grid=(N_ATOMS // NB,),
        in_specs=[pl.BlockSpec((NB, F), lambda i: (i, 0))] * len(partials) + [
            pl.BlockSpec((F, F), lambda i: (0, 0)),
            pl.BlockSpec((1, F), lambda i: (0, 0)),
            pl.BlockSpec((F, F), lambda i: (0, 0)),
            pl.BlockSpec((1, F), lambda i: (0, 0)),
        ],
        out_specs=pl.BlockSpec((NB, F), lambda i: (i, 0)),
        out_shape=jax.ShapeDtypeStruct((N_ATOMS, F), jnp.float32),
    )(*partials, W2, b2.reshape(1, F), W3, b3.reshape(1, F))

    return out
